# Initial kernel scaffold; baseline (speedup 1.0000x reference)
import jax
import jax.numpy as jnp
from jax.experimental import pallas as pl


def kernel(logits, edge_index, vertex):
    raise NotImplementedError("write your pallas kernel here")


# speedup vs baseline: 8.4368x; 8.4368x over previous
import functools

import jax
import jax.numpy as jnp
from jax import lax
from jax.experimental import pallas as pl
from jax.experimental.pallas import tpu as pltpu
from jax.experimental.pallas import tpu_sc as plsc

N_NODES = 50000
N_EDGES = 1600000
NC = 2
NS = 16
L = 16

N_PAD = 50176
HALF = N_PAD // NC
TSPAN = HALF // NS
EPT = N_EDGES // NS
EBLK = 2000
NBLK = EPT // EBLK
LAST_W = N_NODES - (NC * HALF - TSPAN)


def _mask_body(row_hbm, col_hbm, vparam_hbm, out_hbm,
               reach, colbuf, rowbuf, vparam, acc, tmp, outbuf, shared):
    cid = lax.axis_index("c")
    sid = lax.axis_index("s")

    pltpu.sync_copy(vparam_hbm, vparam)
    vtx = vparam[...]
    lo = cid * HALF
    lo_v = jnp.full((L,), lo, dtype=jnp.int32)

    zero_f = jnp.zeros((L,), jnp.float32)
    one_f = jnp.ones((L,), jnp.float32)
    ninf = jnp.full((L,), -jnp.inf, jnp.float32)

    def zbody(i, c):
        reach[pl.ds(i * L, L)] = zero_f
        return c
    lax.fori_loop(0, HALF // L, zbody, 0)

    ebase = sid * EPT

    def blk(b, c):
        off = ebase + b * EBLK
        pltpu.sync_copy(col_hbm.at[pl.ds(off, EBLK)], colbuf)
        pltpu.sync_copy(row_hbm.at[pl.ds(off, EBLK)], rowbuf)

        def step(j, c2):
            cv = colbuf[pl.ds(j * L, L)]
            rv = rowbuf[pl.ds(j * L, L)]
            hit = ((cv == vtx) & (rv != vtx)
                   & (rv >= lo_v) & (rv < lo_v + HALF))
            plsc.store_scatter(reach, [rv - lo_v], one_f, mask=hit)
            return c2
        lax.fori_loop(0, EBLK // L, step, c)
        return c
    lax.fori_loop(0, NBLK, blk, 0)

    pltpu.sync_copy(reach, shared.at[pl.ds(sid * HALF, HALF)])
    plsc.subcore_barrier()

    myoff = sid * TSPAN
    pltpu.sync_copy(shared.at[pl.ds(myoff, TSPAN)], acc)

    def rtile(t, c):
        pltpu.sync_copy(shared.at[pl.ds(t * HALF + myoff, TSPAN)], tmp)

        def av(j, c2):
            s = pl.ds(j * L, L)
            acc[s] = acc[s] + tmp[s]
            return c2
        lax.fori_loop(0, TSPAN // L, av, c)
        return c
    lax.fori_loop(1, NS, rtile, 0)

    neg1 = vtx == jnp.full((L,), -1, dtype=jnp.int32)

    def fv(j, c):
        s = pl.ds(j * L, L)
        a = acc[s]
        o = jnp.where(a > zero_f, zero_f, ninf)
        o = jnp.where(neg1, zero_f, o)
        outbuf[s] = o
        return c
    lax.fori_loop(0, TSPAN // L, fv, 0)

    gbase = lo + myoff
    is_last = (cid == NC - 1) & (sid == NS - 1)

    @pl.when(jnp.logical_not(is_last))
    def _():
        pltpu.sync_copy(outbuf, out_hbm.at[pl.ds(gbase, TSPAN)])

    @pl.when(is_last)
    def _():
        pltpu.sync_copy(outbuf.at[pl.ds(0, LAST_W)],
                        out_hbm.at[pl.ds(gbase, LAST_W)])


_sc_mask = functools.partial(
    pl.kernel,
    mesh=plsc.VectorSubcoreMesh(core_axis_name="c", subcore_axis_name="s"),
    out_type=jax.ShapeDtypeStruct((N_NODES,), jnp.float32),
    compiler_params=pltpu.CompilerParams(needs_layout_passes=False),
    scratch_types=[
        pltpu.VMEM((HALF,), jnp.float32),
        pltpu.VMEM((EBLK,), jnp.int32),
        pltpu.VMEM((EBLK,), jnp.int32),
        pltpu.VMEM((L,), jnp.int32),
        pltpu.VMEM((TSPAN,), jnp.float32),
        pltpu.VMEM((TSPAN,), jnp.float32),
        pltpu.VMEM((TSPAN,), jnp.float32),
        pltpu.VMEM_SHARED((NS * HALF,), jnp.float32),
    ],
)(_mask_body)


def kernel(logits, edge_index, vertex):
    del logits
    row = edge_index[0]
    col = edge_index[1]
    vparam = jnp.full((L,), vertex, dtype=jnp.int32)
    mask = _sc_mask(row, col, vparam)
    return mask.reshape(-1, 1)


# speedup vs baseline: 12.7990x; 1.5170x over previous
import functools

import jax
import jax.numpy as jnp
from jax import lax
from jax.experimental import pallas as pl
from jax.experimental.pallas import tpu as pltpu
from jax.experimental.pallas import tpu_sc as plsc

N_NODES = 50000
N_EDGES = 1600000
NC = 2
NS = 16
L = 16

N_PAD = 50176
HALF = N_PAD // NC
TSPAN = HALF // NS
EPT = N_EDGES // NS
EBLK = 2000
NBLK = EPT // EBLK
NPAIR = NBLK // 2
LAST_W = N_NODES - (NC * HALF - TSPAN)
ZU = 8
SU = 5


def _mask_body(row_hbm, col_hbm, vparam_hbm, out_hbm,
               reach, colA, rowA, colB, rowB, vparam, redbuf, outbuf,
               shared, semA, semB, rsem):
    cid = lax.axis_index("c")
    sid = lax.axis_index("s")
    ebase = sid * EPT

    def start_blk(b, cbuf, rbuf, sem):
        off = ebase + b * EBLK
        pltpu.make_async_copy(col_hbm.at[pl.ds(off, EBLK)], cbuf, sem).start()
        pltpu.make_async_copy(row_hbm.at[pl.ds(off, EBLK)], rbuf, sem).start()

    def wait_blk(cbuf, rbuf, sem):
        pltpu.make_async_copy(col_hbm.at[pl.ds(0, EBLK)], cbuf, sem).wait()
        pltpu.make_async_copy(row_hbm.at[pl.ds(0, EBLK)], rbuf, sem).wait()

    start_blk(0, colA, rowA, semA)
    start_blk(1, colB, rowB, semB)

    pltpu.sync_copy(vparam_hbm, vparam)
    vtx = vparam[...]
    lo = cid * HALF
    lo_v = jnp.full((L,), lo, dtype=jnp.int32)
    hi_v = lo_v + HALF

    zero_f = jnp.zeros((L,), jnp.float32)
    one_f = jnp.ones((L,), jnp.float32)
    ninf = jnp.full((L,), -jnp.inf, jnp.float32)

    def zbody(i, c):
        for u in range(ZU):
            reach[pl.ds((i * ZU + u) * L, L)] = zero_f
        return c
    lax.fori_loop(0, HALF // L // ZU, zbody, 0)

    def scan(cbuf, rbuf):
        def step(j, c):
            for u in range(SU):
                s = pl.ds((j * SU + u) * L, L)
                cv = cbuf[s]
                rv = rbuf[s]
                hit = ((cv == vtx) & (rv != vtx)
                       & (rv >= lo_v) & (rv < hi_v))
                plsc.store_scatter(reach, [rv - lo_v], one_f, mask=hit)
            return c
        lax.fori_loop(0, EBLK // L // SU, step, 0)

    def pair(p, c):
        wait_blk(colA, rowA, semA)
        scan(colA, rowA)

        @pl.when(p < NPAIR - 1)
        def _():
            start_blk(2 * p + 2, colA, rowA, semA)

        wait_blk(colB, rowB, semB)
        scan(colB, rowB)

        @pl.when(p < NPAIR - 1)
        def _():
            start_blk(2 * p + 3, colB, rowB, semB)
        return c
    lax.fori_loop(0, NPAIR, pair, 0)

    pltpu.sync_copy(reach, shared.at[pl.ds(sid * HALF, HALF)])
    plsc.subcore_barrier()

    myoff = sid * TSPAN
    for t in range(NS):
        pltpu.make_async_copy(shared.at[pl.ds(t * HALF + myoff, TSPAN)],
                              redbuf.at[pl.ds(t * TSPAN, TSPAN)],
                              rsem).start()
    for t in range(NS):
        pltpu.make_async_copy(shared.at[pl.ds(myoff, TSPAN)],
                              redbuf.at[pl.ds(t * TSPAN, TSPAN)],
                              rsem).wait()

    neg1 = vtx == jnp.full((L,), -1, dtype=jnp.int32)

    def fv(j, c):
        s0 = pl.ds(j * L, L)
        a = redbuf[s0]
        for t in range(1, NS):
            a = a + redbuf[pl.ds(t * TSPAN + j * L, L)]
        o = jnp.where(a > zero_f, zero_f, ninf)
        o = jnp.where(neg1, zero_f, o)
        outbuf[s0] = o
        return c
    lax.fori_loop(0, TSPAN // L, fv, 0)

    gbase = lo + myoff
    is_last = (cid == NC - 1) & (sid == NS - 1)

    @pl.when(jnp.logical_not(is_last))
    def _():
        pltpu.sync_copy(outbuf, out_hbm.at[pl.ds(gbase, TSPAN)])

    @pl.when(is_last)
    def _():
        pltpu.sync_copy(outbuf.at[pl.ds(0, LAST_W)],
                        out_hbm.at[pl.ds(gbase, LAST_W)])


_sc_mask = functools.partial(
    pl.kernel,
    mesh=plsc.VectorSubcoreMesh(core_axis_name="c", subcore_axis_name="s"),
    out_type=jax.ShapeDtypeStruct((N_NODES,), jnp.float32),
    compiler_params=pltpu.CompilerParams(needs_layout_passes=False),
    scratch_types=[
        pltpu.VMEM((HALF,), jnp.float32),
        pltpu.VMEM((EBLK,), jnp.int32),
        pltpu.VMEM((EBLK,), jnp.int32),
        pltpu.VMEM((EBLK,), jnp.int32),
        pltpu.VMEM((EBLK,), jnp.int32),
        pltpu.VMEM((L,), jnp.int32),
        pltpu.VMEM((NS * TSPAN,), jnp.float32),
        pltpu.VMEM((TSPAN,), jnp.float32),
        pltpu.VMEM_SHARED((NS * HALF,), jnp.float32),
        pltpu.SemaphoreType.DMA,
        pltpu.SemaphoreType.DMA,
        pltpu.SemaphoreType.DMA,
    ],
)(_mask_body)


def kernel(logits, edge_index, vertex):
    del logits
    row = edge_index[0]
    col = edge_index[1]
    vparam = jnp.full((L,), vertex, dtype=jnp.int32)
    mask = _sc_mask(row, col, vparam)
    return mask.reshape(-1, 1)


# speedup vs baseline: 13.0933x; 1.0230x over previous
import functools

import jax
import jax.numpy as jnp
from jax import lax
from jax.experimental import pallas as pl
from jax.experimental.pallas import tpu as pltpu
from jax.experimental.pallas import tpu_sc as plsc

N_NODES = 50000
N_EDGES = 1600000
NC = 2
NS = 16
L = 16

N_PAD = 50176
HALF = N_PAD // NC
TSPAN = HALF // NS
EPT = N_EDGES // NS
EBLK = 10000
NBLK = EPT // EBLK
NPAIR = NBLK // 2
LAST_W = N_NODES - (NC * HALF - TSPAN)
ZU = 8
SU = 5


def _mask_body(row_hbm, col_hbm, vparam_hbm, out_hbm,
               reach, colA, rowA, colB, rowB, vparam, redbuf, outbuf,
               shared, semA, semB, rsem):
    cid = lax.axis_index("c")
    sid = lax.axis_index("s")
    ebase = sid * EPT

    def start_blk(b, cbuf, rbuf, sem):
        off = ebase + b * EBLK
        pltpu.make_async_copy(col_hbm.at[pl.ds(off, EBLK)], cbuf, sem).start()
        pltpu.make_async_copy(row_hbm.at[pl.ds(off, EBLK)], rbuf, sem).start()

    def wait_blk(cbuf, rbuf, sem):
        pltpu.make_async_copy(col_hbm.at[pl.ds(0, EBLK)], cbuf, sem).wait()
        pltpu.make_async_copy(row_hbm.at[pl.ds(0, EBLK)], rbuf, sem).wait()

    start_blk(0, colA, rowA, semA)
    start_blk(1, colB, rowB, semB)

    pltpu.sync_copy(vparam_hbm, vparam)
    vtx = vparam[...]
    lo = cid * HALF
    lo_v = jnp.full((L,), lo, dtype=jnp.int32)
    half_u = jnp.full((L,), HALF, dtype=jnp.uint32)

    zero_f = jnp.zeros((L,), jnp.float32)
    one_f = jnp.ones((L,), jnp.float32)
    ninf = jnp.full((L,), -jnp.inf, jnp.float32)

    def zbody(i, c):
        for u in range(ZU):
            reach[pl.ds((i * ZU + u) * L, L)] = zero_f
        return c
    lax.fori_loop(0, HALF // L // ZU, zbody, 0)

    def scan(cbuf, rbuf):
        def step(j, c):
            for u in range(SU):
                s = pl.ds((j * SU + u) * L, L)
                cv = cbuf[s]
                rv = rbuf[s]
                d = rv - lo_v
                in_half = plsc.bitcast(d, jnp.uint32) < half_u
                hit = (cv == vtx) & (rv != vtx) & in_half
                plsc.store_scatter(reach, [d], one_f, mask=hit)
            return c
        lax.fori_loop(0, EBLK // L // SU, step, 0)

    def pair(p, c):
        wait_blk(colA, rowA, semA)
        scan(colA, rowA)

        @pl.when(p < NPAIR - 1)
        def _():
            start_blk(2 * p + 2, colA, rowA, semA)

        wait_blk(colB, rowB, semB)
        scan(colB, rowB)

        @pl.when(p < NPAIR - 1)
        def _():
            start_blk(2 * p + 3, colB, rowB, semB)
        return c
    lax.fori_loop(0, NPAIR, pair, 0)

    pltpu.sync_copy(reach, shared.at[pl.ds(sid * HALF, HALF)])
    plsc.subcore_barrier()

    myoff = sid * TSPAN
    for t in range(NS):
        pltpu.make_async_copy(shared.at[pl.ds(t * HALF + myoff, TSPAN)],
                              redbuf.at[pl.ds(t * TSPAN, TSPAN)],
                              rsem).start()
    for t in range(NS):
        pltpu.make_async_copy(shared.at[pl.ds(myoff, TSPAN)],
                              redbuf.at[pl.ds(t * TSPAN, TSPAN)],
                              rsem).wait()

    neg1 = vtx == jnp.full((L,), -1, dtype=jnp.int32)

    def fv(j, c):
        s0 = pl.ds(j * L, L)
        a = redbuf[s0]
        for t in range(1, NS):
            a = a + redbuf[pl.ds(t * TSPAN + j * L, L)]
        o = jnp.where(a > zero_f, zero_f, ninf)
        o = jnp.where(neg1, zero_f, o)
        outbuf[s0] = o
        return c
    lax.fori_loop(0, TSPAN // L, fv, 0)

    gbase = lo + myoff
    is_last = (cid == NC - 1) & (sid == NS - 1)

    @pl.when(jnp.logical_not(is_last))
    def _():
        pltpu.sync_copy(outbuf, out_hbm.at[pl.ds(gbase, TSPAN)])

    @pl.when(is_last)
    def _():
        pltpu.sync_copy(outbuf.at[pl.ds(0, LAST_W)],
                        out_hbm.at[pl.ds(gbase, LAST_W)])


_sc_mask = functools.partial(
    pl.kernel,
    mesh=plsc.VectorSubcoreMesh(core_axis_name="c", subcore_axis_name="s"),
    out_type=jax.ShapeDtypeStruct((N_NODES,), jnp.float32),
    compiler_params=pltpu.CompilerParams(needs_layout_passes=False),
    scratch_types=[
        pltpu.VMEM((HALF,), jnp.float32),
        pltpu.VMEM((EBLK,), jnp.int32),
        pltpu.VMEM((EBLK,), jnp.int32),
        pltpu.VMEM((EBLK,), jnp.int32),
        pltpu.VMEM((EBLK,), jnp.int32),
        pltpu.VMEM((L,), jnp.int32),
        pltpu.VMEM((NS * TSPAN,), jnp.float32),
        pltpu.VMEM((TSPAN,), jnp.float32),
        pltpu.VMEM_SHARED((NS * HALF,), jnp.float32),
        pltpu.SemaphoreType.DMA,
        pltpu.SemaphoreType.DMA,
        pltpu.SemaphoreType.DMA,
    ],
)(_mask_body)


def kernel(logits, edge_index, vertex):
    del logits
    row = edge_index[0]
    col = edge_index[1]
    vparam = jnp.full((L,), vertex, dtype=jnp.int32)
    mask = _sc_mask(row, col, vparam)
    return mask.reshape(-1, 1)
